# Initial kernel scaffold; baseline (speedup 1.0000x reference)
import jax
import jax.numpy as jnp
from jax.experimental import pallas as pl


def kernel(adj, weight, features, W_feat, b_feat, W_msg, centroids, W_out, b_out):
    raise NotImplementedError("write your pallas kernel here")


# speedup vs baseline: 1.8585x; 1.8585x over previous
import functools

import jax
import jax.numpy as jnp
from jax import lax
from jax.experimental import pallas as pl
from jax.experimental.pallas import tpu as pltpu
from jax.experimental.pallas import tpu_sc as plsc

N_NODES = 10000
MAXN = 32
DIM = 128
N_CENT = 100
N_CLASS = 40

NC = 2
NS = 16
NW = NC * NS
NPAD = 10240
NPW = NPAD // NW
NODES_PER_XFER = 128 // MAXN
TPW = NPW // NODES_PER_XFER
IDX_ROWS = NPAD * MAXN // 128

BLK = 256
GRID = NPAD // BLK

_f32 = jnp.float32


def _projmsg_body(x_ref, wf_ref, b_ref, wm_ref, o_ref):
    h = jnp.dot(x_ref[...], wf_ref[...], preferred_element_type=_f32)
    h = jnp.maximum(h + b_ref[...], 0.0)
    o_ref[...] = jnp.dot(h, wm_ref[...], preferred_element_type=_f32)


def _tc_projmsg(feat, W_feat, b_feat2, W_msg):
    return pl.pallas_call(
        _projmsg_body,
        grid=(GRID,),
        in_specs=[
            pl.BlockSpec((BLK, DIM), lambda i: (i, 0)),
            pl.BlockSpec((DIM, DIM), lambda i: (0, 0)),
            pl.BlockSpec((1, DIM), lambda i: (0, 0)),
            pl.BlockSpec((DIM, DIM), lambda i: (0, 0)),
        ],
        out_specs=pl.BlockSpec((BLK, DIM), lambda i: (i, 0)),
        out_shape=jax.ShapeDtypeStruct((NPAD, DIM), _f32),
    )(feat, W_feat, b_feat2, W_msg)


def _msg_body(x_ref, wm_ref, o_ref):
    o_ref[...] = jnp.dot(x_ref[...], wm_ref[...], preferred_element_type=_f32)


def _tc_msg(nr, W_msg):
    return pl.pallas_call(
        _msg_body,
        grid=(GRID,),
        in_specs=[
            pl.BlockSpec((BLK, DIM), lambda i: (i, 0)),
            pl.BlockSpec((DIM, DIM), lambda i: (0, 0)),
        ],
        out_specs=pl.BlockSpec((BLK, DIM), lambda i: (i, 0)),
        out_shape=jax.ShapeDtypeStruct((NPAD, DIM), _f32),
    )(nr, W_msg)


def _head_body(nr_ref, centT_ref, wout_ref, bout_ref, logp_ref):
    nr = nr_ref[...]
    centT = centT_ref[...]
    r2 = jnp.sum(nr * nr, axis=1, keepdims=True)
    c2 = jnp.sum(centT * centT, axis=0, keepdims=True)
    rc = jnp.dot(nr, centT, preferred_element_type=_f32)
    sq = r2 + c2 - 2.0 * rc
    sim = jnp.sqrt(jnp.maximum(sq, 1e-12))
    logit = jnp.dot(sim, wout_ref[...], preferred_element_type=_f32) + bout_ref[...]
    m = jnp.max(logit, axis=1, keepdims=True)
    e = jnp.exp(logit - m)
    lse = jnp.log(jnp.sum(e, axis=1, keepdims=True)) + m
    logp_ref[...] = logit - lse


def _tc_head(nr, centT, W_out_p, b_out2):
    return pl.pallas_call(
        _head_body,
        grid=(GRID,),
        in_specs=[
            pl.BlockSpec((BLK, DIM), lambda i: (i, 0)),
            pl.BlockSpec((DIM, 128), lambda i: (0, 0)),
            pl.BlockSpec((128, N_CLASS), lambda i: (0, 0)),
            pl.BlockSpec((1, N_CLASS), lambda i: (0, 0)),
        ],
        out_specs=pl.BlockSpec((BLK, N_CLASS), lambda i: (i, 0)),
        out_shape=jax.ShapeDtypeStruct((NPAD, N_CLASS), _f32),
    )(nr, centT, W_out_p, b_out2)


_GDN = lax.GatherDimensionNumbers(
    offset_dims=(), collapsed_slice_dims=(0,), start_index_map=(0,))


def _lane_bcast(vec, k):
    idx = jnp.full((16, 1), k, dtype=jnp.int32)
    return lax.gather(vec, idx, _GDN, (1,),
                      mode=lax.GatherScatterMode.PROMISE_IN_BOUNDS)


def _sc_body(adj_hbm, wt_hbm, msg_hbm, out_hbm,
             idx_v, wt_v, rbuf0, rbuf1, out_v, sem0, sem1):
    wid = lax.axis_index("s") * NC + lax.axis_index("c")
    tbase = wid * TPW
    nbase = wid * NPW

    pltpu.sync_copy(adj_hbm.at[pl.ds(tbase, TPW)], idx_v)
    pltpu.sync_copy(wt_hbm.at[pl.ds(tbase, TPW)], wt_v)

    def compute(t, buf):
        for j in range(NODES_PER_XFER):
            w_lo = wt_v[t, pl.ds(j * MAXN, 16)]
            w_hi = wt_v[t, pl.ds(j * MAXN + 16, 16)]
            acc = tuple(jnp.zeros((16,), _f32) for _ in range(8))

            def mk(base_row, wv):
                def body(k, a):
                    wk = _lane_bcast(wv, k)
                    row = base_row + k
                    return tuple(a[c] + wk * buf[row, pl.ds(c * 16, 16)]
                                 for c in range(8))
                return body

            acc = lax.fori_loop(0, 16, mk(j * MAXN, w_lo), acc, unroll=4)
            acc = lax.fori_loop(0, 16, mk(j * MAXN + 16, w_hi), acc, unroll=4)
            n = t * NODES_PER_XFER + j
            for c in range(8):
                out_v[n, pl.ds(c * 16, 16)] = jnp.maximum(acc[c], 0.0)

    pltpu.async_copy(msg_hbm.at[idx_v.at[0]], rbuf0, sem0)

    def pair(i, carry):
        t0 = 2 * i
        t1 = 2 * i + 1
        pltpu.async_copy(msg_hbm.at[idx_v.at[t1]], rbuf1, sem1)
        pltpu.make_async_copy(msg_hbm.at[idx_v.at[t0]], rbuf0, sem0).wait()
        compute(t0, rbuf0)

        @pl.when(i < TPW // 2 - 1)
        def _():
            pltpu.async_copy(msg_hbm.at[idx_v.at[t0 + 2]], rbuf0, sem0)

        pltpu.make_async_copy(msg_hbm.at[idx_v.at[t1]], rbuf1, sem1).wait()
        compute(t1, rbuf1)
        return carry

    lax.fori_loop(0, TPW // 2, pair, 0)

    pltpu.sync_copy(out_v, out_hbm.at[pl.ds(nbase, NPW)])


@functools.partial(
    pl.kernel,
    mesh=plsc.VectorSubcoreMesh(core_axis_name="c", subcore_axis_name="s"),
    out_type=jax.ShapeDtypeStruct((NPAD, DIM), _f32),
    scratch_types=[
        pltpu.VMEM((TPW, 128), jnp.int32),
        pltpu.VMEM((TPW, 128), _f32),
        pltpu.VMEM((128, DIM), _f32),
        pltpu.VMEM((128, DIM), _f32),
        pltpu.VMEM((NPW, DIM), _f32),
        pltpu.SemaphoreType.DMA,
        pltpu.SemaphoreType.DMA,
    ],
)
def _sc_combine(adj_hbm, wt_hbm, msg_hbm, out_hbm,
                idx_v, wt_v, rbuf0, rbuf1, out_v, sem0, sem1):
    _sc_body(adj_hbm, wt_hbm, msg_hbm, out_hbm,
             idx_v, wt_v, rbuf0, rbuf1, out_v, sem0, sem1)


def kernel(adj, weight, features, W_feat, b_feat, W_msg, centroids, W_out, b_out):
    pad = NPAD - N_NODES
    adjp = jnp.pad(adj[0].astype(jnp.int32), ((0, pad), (0, 0)))
    adjp = adjp.reshape(IDX_ROWS, 128)
    wtp = jnp.pad(weight[0], ((0, pad), (0, 0))).reshape(IDX_ROWS, 128)
    featp = jnp.pad(features[0], ((0, pad), (0, 0)))
    b_feat2 = b_feat.reshape(1, DIM)
    centT = jnp.pad(centroids, ((0, 128 - N_CENT), (0, 0))).T
    W_out_p = jnp.pad(W_out, ((0, 128 - N_CENT), (0, 0)))
    b_out2 = b_out.reshape(1, N_CLASS)

    msg0 = _tc_projmsg(featp, W_feat, b_feat2, W_msg)
    nr1 = _sc_combine(adjp, wtp, msg0)
    msg1 = _tc_msg(nr1, W_msg)
    nr2 = _sc_combine(adjp, wtp, msg1)
    logp = _tc_head(nr2, centT, W_out_p, b_out2)
    return (logp[:N_NODES], nr2[:N_NODES])


# speedup vs baseline: 1.9005x; 1.0226x over previous
import functools

import jax
import jax.numpy as jnp
from jax import lax
from jax.experimental import pallas as pl
from jax.experimental.pallas import tpu as pltpu
from jax.experimental.pallas import tpu_sc as plsc

N_NODES = 10000
MAXN = 32
DIM = 128
N_CENT = 100
N_CLASS = 40

NC = 2
NS = 16
NW = NC * NS
NPAD = 10240
NPW = NPAD // NW
NODES_PER_XFER = 128 // MAXN
TPW = NPW // NODES_PER_XFER
IDX_ROWS = NPAD * MAXN // 128

BLK = 256
GRID = NPAD // BLK

_f32 = jnp.float32


def _projmsg_body(x_ref, wf_ref, b_ref, wm_ref, o_ref):
    h = jnp.dot(x_ref[...], wf_ref[...], preferred_element_type=_f32)
    h = jnp.maximum(h + b_ref[...], 0.0)
    o_ref[...] = jnp.dot(h, wm_ref[...], preferred_element_type=_f32)


def _tc_projmsg(feat, W_feat, b_feat2, W_msg):
    return pl.pallas_call(
        _projmsg_body,
        grid=(GRID,),
        in_specs=[
            pl.BlockSpec((BLK, DIM), lambda i: (i, 0)),
            pl.BlockSpec((DIM, DIM), lambda i: (0, 0)),
            pl.BlockSpec((1, DIM), lambda i: (0, 0)),
            pl.BlockSpec((DIM, DIM), lambda i: (0, 0)),
        ],
        out_specs=pl.BlockSpec((BLK, DIM), lambda i: (i, 0)),
        out_shape=jax.ShapeDtypeStruct((NPAD, DIM), _f32),
    )(feat, W_feat, b_feat2, W_msg)


def _msg_body(x_ref, wm_ref, o_ref):
    o_ref[...] = jnp.dot(x_ref[...], wm_ref[...], preferred_element_type=_f32)


def _tc_msg(nr, W_msg):
    return pl.pallas_call(
        _msg_body,
        grid=(GRID,),
        in_specs=[
            pl.BlockSpec((BLK, DIM), lambda i: (i, 0)),
            pl.BlockSpec((DIM, DIM), lambda i: (0, 0)),
        ],
        out_specs=pl.BlockSpec((BLK, DIM), lambda i: (i, 0)),
        out_shape=jax.ShapeDtypeStruct((NPAD, DIM), _f32),
    )(nr, W_msg)


def _head_body(nr_ref, centT_ref, wout_ref, bout_ref, logp_ref):
    nr = nr_ref[...]
    centT = centT_ref[...]
    r2 = jnp.sum(nr * nr, axis=1, keepdims=True)
    c2 = jnp.sum(centT * centT, axis=0, keepdims=True)
    rc = jnp.dot(nr, centT, preferred_element_type=_f32)
    sq = r2 + c2 - 2.0 * rc
    sim = jnp.sqrt(jnp.maximum(sq, 1e-12))
    logit = jnp.dot(sim, wout_ref[...], preferred_element_type=_f32) + bout_ref[...]
    m = jnp.max(logit, axis=1, keepdims=True)
    e = jnp.exp(logit - m)
    lse = jnp.log(jnp.sum(e, axis=1, keepdims=True)) + m
    logp_ref[...] = logit - lse


def _tc_head(nr, centT, W_out_p, b_out2):
    return pl.pallas_call(
        _head_body,
        grid=(GRID,),
        in_specs=[
            pl.BlockSpec((BLK, DIM), lambda i: (i, 0)),
            pl.BlockSpec((DIM, 128), lambda i: (0, 0)),
            pl.BlockSpec((128, N_CLASS), lambda i: (0, 0)),
            pl.BlockSpec((1, N_CLASS), lambda i: (0, 0)),
        ],
        out_specs=pl.BlockSpec((BLK, N_CLASS), lambda i: (i, 0)),
        out_shape=jax.ShapeDtypeStruct((NPAD, N_CLASS), _f32),
    )(nr, centT, W_out_p, b_out2)


_GDN = lax.GatherDimensionNumbers(
    offset_dims=(), collapsed_slice_dims=(0,), start_index_map=(0,))


def _lane_bcast(vec, k):
    idx = jnp.full((16, 1), k, dtype=jnp.int32)
    return lax.gather(vec, idx, _GDN, (1,),
                      mode=lax.GatherScatterMode.PROMISE_IN_BOUNDS)


NRING = 4


def _sc_body(adj_hbm, wt_hbm, msg_hbm, out_hbm,
             idx_v, wt_v, rbuf, out_v, sems):
    wid = lax.axis_index("s") * NC + lax.axis_index("c")
    tbase = wid * TPW
    nbase = wid * NPW

    pltpu.sync_copy(adj_hbm.at[pl.ds(tbase, TPW)], idx_v)
    pltpu.sync_copy(wt_hbm.at[pl.ds(tbase, TPW)], wt_v)

    for s in range(NRING):
        pltpu.async_copy(msg_hbm.at[idx_v.at[s]], rbuf.at[s], sems.at[s])

    def body(t, carry):
        slot = lax.rem(t, NRING)
        pltpu.make_async_copy(msg_hbm.at[idx_v.at[t]], rbuf.at[slot],
                              sems.at[slot]).wait()
        buf = rbuf.at[slot]
        for j in range(NODES_PER_XFER):
            w_lo = wt_v[t, pl.ds(j * MAXN, 16)]
            w_hi = wt_v[t, pl.ds(j * MAXN + 16, 16)]
            acc = [jnp.zeros((16,), _f32) for _ in range(8)]
            for k in range(MAXN):
                wk = _lane_bcast(w_lo if k < 16 else w_hi, k % 16)
                row = j * MAXN + k
                for c in range(8):
                    acc[c] = acc[c] + wk * buf[row, pl.ds(c * 16, 16)]
            n = t * NODES_PER_XFER + j
            for c in range(8):
                out_v[n, pl.ds(c * 16, 16)] = jnp.maximum(acc[c], 0.0)

        @pl.when(t + NRING < TPW)
        def _():
            pltpu.async_copy(msg_hbm.at[idx_v.at[t + NRING]], rbuf.at[slot],
                             sems.at[slot])

        return carry

    lax.fori_loop(0, TPW, body, 0)

    pltpu.sync_copy(out_v, out_hbm.at[pl.ds(nbase, NPW)])


@functools.partial(
    pl.kernel,
    mesh=plsc.VectorSubcoreMesh(core_axis_name="c", subcore_axis_name="s"),
    out_type=jax.ShapeDtypeStruct((NPAD, DIM), _f32),
    scratch_types=[
        pltpu.VMEM((TPW, 128), jnp.int32),
        pltpu.VMEM((TPW, 128), _f32),
        pltpu.VMEM((NRING, 128, DIM), _f32),
        pltpu.VMEM((NPW, DIM), _f32),
        pltpu.SemaphoreType.DMA((NRING,)),
    ],
)
def _sc_combine(adj_hbm, wt_hbm, msg_hbm, out_hbm,
                idx_v, wt_v, rbuf, out_v, sems):
    _sc_body(adj_hbm, wt_hbm, msg_hbm, out_hbm,
             idx_v, wt_v, rbuf, out_v, sems)


def kernel(adj, weight, features, W_feat, b_feat, W_msg, centroids, W_out, b_out):
    pad = NPAD - N_NODES
    adjp = jnp.pad(adj[0].astype(jnp.int32), ((0, pad), (0, 0)))
    adjp = adjp.reshape(IDX_ROWS, 128)
    wtp = jnp.pad(weight[0], ((0, pad), (0, 0))).reshape(IDX_ROWS, 128)
    featp = jnp.pad(features[0], ((0, pad), (0, 0)))
    b_feat2 = b_feat.reshape(1, DIM)
    centT = jnp.pad(centroids, ((0, 128 - N_CENT), (0, 0))).T
    W_out_p = jnp.pad(W_out, ((0, 128 - N_CENT), (0, 0)))
    b_out2 = b_out.reshape(1, N_CLASS)

    msg0 = _tc_projmsg(featp, W_feat, b_feat2, W_msg)
    nr1 = _sc_combine(adjp, wtp, msg0)
    msg1 = _tc_msg(nr1, W_msg)
    nr2 = _sc_combine(adjp, wtp, msg1)
    logp = _tc_head(nr2, centT, W_out_p, b_out2)
    return (logp[:N_NODES], nr2[:N_NODES])
